# baseline (speedup 1.0000x reference)
import functools
import math

import jax
import jax.numpy as jnp
from jax import lax
from jax.experimental import pallas as pl
from jax.experimental.pallas import tpu as pltpu
from jax.experimental.pallas import tpu_sc as plsc

D_MODEL = 64
SCALE = math.sqrt(D_MODEL)

_NC = 2
_NS = 16
_NW = _NC * _NS
_LANES = 16
_CH = 128
_NBUF = 2


@functools.lru_cache(maxsize=None)
def _make_gather_kernel(n_rows: int, vocab_pairs: int):
    assert n_rows % (_NW * _CH * _NBUF) == 0
    per_w = n_rows // _NW
    n_outer = per_w // (_CH * _NBUF)

    mesh = plsc.VectorSubcoreMesh(core_axis_name="c", subcore_axis_name="s")

    scratch = (
        [pltpu.VMEM((_CH,), jnp.int32) for _ in range(2 * _NBUF)]
        + [pltpu.VMEM((_CH, 2 * D_MODEL), jnp.float32) for _ in range(_NBUF)]
        + [pltpu.VMEM((_CH, D_MODEL), jnp.float32) for _ in range(_NBUF)]
        + [pltpu.SemaphoreType.DMA] * (4 * _NBUF)
    )

    @functools.partial(
        pl.kernel,
        out_type=jax.ShapeDtypeStruct((n_rows, D_MODEL), jnp.float32),
        mesh=mesh,
        scratch_types=scratch,
    )
    def gather_scale(tab_hbm, iq_hbm, cb_hbm, out_hbm, *sc):
        iq_v = sc[:_NBUF]
        cb_v = sc[_NBUF:2 * _NBUF]
        gat_v = sc[2 * _NBUF:3 * _NBUF]
        slab_v = sc[3 * _NBUF:4 * _NBUF]
        iq_s = sc[4 * _NBUF:5 * _NBUF]
        cb_s = sc[5 * _NBUF:6 * _NBUF]
        gat_s = sc[6 * _NBUF:7 * _NBUF]
        out_s = sc[7 * _NBUF:8 * _NBUF]

        wid = lax.axis_index("s") * _NC + lax.axis_index("c")
        base = wid * per_w

        @pl.loop(0, n_outer)
        def _outer(o):
            c0 = base + o * (_NBUF * _CH)

            idx_dma = [
                (pltpu.async_copy(
                    iq_hbm.at[pl.ds(c0 + b * _CH, _CH)], iq_v[b], iq_s[b]),
                 pltpu.async_copy(
                    cb_hbm.at[pl.ds(c0 + b * _CH, _CH)], cb_v[b], cb_s[b]))
                for b in range(_NBUF)
            ]

            gat_dma = []
            for b in range(_NBUF):
                idx_dma[b][0].wait()

                gat_dma.append(
                    pltpu.async_copy(tab_hbm.at[iq_v[b]], gat_v[b], gat_s[b]))

            for b in range(_NBUF):
                gat_dma[b].wait()
                idx_dma[b][1].wait()

                @pl.when(o > 0)
                def _drain():
                    pltpu.make_async_copy(
                        slab_v[b], out_hbm.at[pl.ds(0, _CH)], out_s[b]).wait()

                @pl.loop(0, _CH // _LANES)
                def _select(g):
                    cbv = cb_v[b][pl.ds(g * _LANES, _LANES)]
                    for l in range(_LANES):
                        r = g * _LANES + l
                        cbase = cbv[l]
                        for d in range(D_MODEL // _LANES):
                            vals = gat_v[b][
                                r, pl.ds(cbase + d * _LANES, _LANES)]
                            slab_v[b][r, pl.ds(d * _LANES, _LANES)] = (
                                vals * SCALE)

                pltpu.async_copy(
                    slab_v[b], out_hbm.at[pl.ds(c0 + b * _CH, _CH)], out_s[b])

        for b in range(_NBUF):
            pltpu.make_async_copy(
                slab_v[b], out_hbm.at[pl.ds(0, _CH)], out_s[b]).wait()

    return gather_scale


def kernel(x, lut):
    b, s = x.shape
    vocab, d = lut.shape
    flat = x.reshape(b * s).astype(jnp.int32)
    iq = flat >> 1
    cb = (flat & 1) << 6
    tab = lut.reshape(vocab // 2, 2 * d)
    out = _make_gather_kernel(b * s, vocab // 2)(tab, iq, cb)
    return out.reshape(b, s, d)


# speedup vs baseline: 1.0922x; 1.0922x over previous
import functools
import math

import jax
import jax.numpy as jnp
from jax import lax
from jax.experimental import pallas as pl
from jax.experimental.pallas import tpu as pltpu
from jax.experimental.pallas import tpu_sc as plsc

D_MODEL = 64
SCALE = math.sqrt(D_MODEL)

_NC = 2
_NS = 16
_NW = _NC * _NS
_LANES = 16
_CH = 128
_NSL = 4
_NG = 2


@functools.lru_cache(maxsize=None)
def _make_gather_kernel(n_rows: int, vocab_pairs: int):
    assert n_rows % (_NW * _CH * _NSL) == 0
    per_w = n_rows // _NW
    n_outer = per_w // (_CH * _NSL)

    mesh = plsc.VectorSubcoreMesh(core_axis_name="c", subcore_axis_name="s")

    scratch = (
        [pltpu.VMEM((_CH,), jnp.int32) for _ in range(2 * _NSL)]
        + [pltpu.VMEM((_CH, 2 * D_MODEL), jnp.float32) for _ in range(_NG)]
        + [pltpu.VMEM((_CH, D_MODEL), jnp.float32) for _ in range(_NSL)]
        + [pltpu.SemaphoreType.DMA] * (2 * _NSL + _NG + _NSL)
    )

    @functools.partial(
        pl.kernel,
        out_type=jax.ShapeDtypeStruct((n_rows, D_MODEL), jnp.float32),
        mesh=mesh,
        scratch_types=scratch,
    )
    def gather_scale(tab_hbm, iq_hbm, cb_hbm, out_hbm, *sc):
        iq_v = sc[:_NSL]
        cb_v = sc[_NSL:2 * _NSL]
        gat_v = sc[2 * _NSL:2 * _NSL + _NG]
        slab_v = sc[2 * _NSL + _NG:2 * _NSL + _NG + _NSL]
        sems = sc[2 * _NSL + _NG + _NSL:]
        iq_s = sems[:_NSL]
        cb_s = sems[_NSL:2 * _NSL]
        gat_s = sems[2 * _NSL:2 * _NSL + _NG]
        out_s = sems[2 * _NSL + _NG:]

        wid = lax.axis_index("s") * _NC + lax.axis_index("c")
        base = wid * per_w

        for b in range(_NSL):
            pltpu.async_copy(
                iq_hbm.at[pl.ds(base + b * _CH, _CH)], iq_v[b], iq_s[b])
            pltpu.async_copy(
                cb_hbm.at[pl.ds(base + b * _CH, _CH)], cb_v[b], cb_s[b])

        @pl.loop(0, n_outer)
        def _outer(o):
            c0 = base + o * (_NSL * _CH)

            for b in range(_NG):
                pltpu.make_async_copy(
                    iq_hbm.at[pl.ds(c0 + b * _CH, _CH)], iq_v[b],
                    iq_s[b]).wait()
                pltpu.async_copy(tab_hbm.at[iq_v[b]], gat_v[b], gat_s[b])

            for b in range(_NSL):
                g = b % _NG
                pltpu.make_async_copy(
                    tab_hbm.at[iq_v[b]], gat_v[g], gat_s[g]).wait()
                pltpu.make_async_copy(
                    cb_hbm.at[pl.ds(c0 + b * _CH, _CH)], cb_v[b],
                    cb_s[b]).wait()

                @pl.when(o > 0)
                def _drain():
                    pltpu.make_async_copy(
                        slab_v[b], out_hbm.at[pl.ds(0, _CH)], out_s[b]).wait()

                @pl.loop(0, _CH // _LANES)
                def _select(gi):
                    cbv = cb_v[b][pl.ds(gi * _LANES, _LANES)]
                    for l in range(_LANES):
                        r = gi * _LANES + l
                        cbase = cbv[l]
                        for d in range(D_MODEL // _LANES):
                            vals = gat_v[g][
                                r, pl.ds(cbase + d * _LANES, _LANES)]
                            slab_v[b][r, pl.ds(d * _LANES, _LANES)] = (
                                vals * SCALE)

                pltpu.async_copy(
                    slab_v[b], out_hbm.at[pl.ds(c0 + b * _CH, _CH)], out_s[b])

                if b + _NG < _NSL:
                    b2 = b + _NG
                    pltpu.make_async_copy(
                        iq_hbm.at[pl.ds(c0 + b2 * _CH, _CH)], iq_v[b2],
                        iq_s[b2]).wait()
                    pltpu.async_copy(tab_hbm.at[iq_v[b2]], gat_v[g], gat_s[g])

                @pl.when(o < n_outer - 1)
                def _prefetch():
                    nxt = c0 + _NSL * _CH + b * _CH
                    pltpu.async_copy(
                        iq_hbm.at[pl.ds(nxt, _CH)], iq_v[b], iq_s[b])
                    pltpu.async_copy(
                        cb_hbm.at[pl.ds(nxt, _CH)], cb_v[b], cb_s[b])

        for b in range(_NSL):
            pltpu.make_async_copy(
                slab_v[b], out_hbm.at[pl.ds(0, _CH)], out_s[b]).wait()

    return gather_scale


def kernel(x, lut):
    b, s = x.shape
    vocab, d = lut.shape
    flat = x.reshape(b * s).astype(jnp.int32)
    iq = flat >> 1
    cb = (flat & 1) << 6
    tab = lut.reshape(vocab // 2, 2 * d)
    out = _make_gather_kernel(b * s, vocab // 2)(tab, iq, cb)
    return out.reshape(b, s, d)
